# baseline (speedup 1.0000x reference)
import functools

import jax
import jax.numpy as jnp
from jax import lax
from jax.experimental import pallas as pl
from jax.experimental.pallas import tpu as pltpu
from jax.experimental.pallas import tpu_sc as plsc

_N = 100000
_D = 128
_G = 128
_B = 256

_NSC = 48000
_S = 128
_NSUB = _NSC // _S
_NW = 32
_MAXJ = -(-_NSUB // _NW)
_HALF = _MAXJ // 2

_RB = 1000
_TC0 = _NSC // _RB
_NTB = (_N - _NSC) // _RB


def _sc_body(x_hbm, batch_hbm, acc_out,
             xbufa, xbufb, ida, idb, zrow, acc_sh, sema, semb):
    cid = lax.axis_index("c")
    sid = lax.axis_index("s")
    wid = sid * 2 + cid

    zero16 = jnp.zeros((16,), jnp.float32)

    def _init_z(i, carry):
        for g in range(_D // 16):
            zrow[i, pl.ds(g * 16, 16)] = zero16
        return carry

    lax.fori_loop(0, 16, _init_z, 0)

    pltpu.sync_copy(zrow, acc_sh.at[pl.ds(sid * 16, 16), :])
    plsc.subcore_barrier()

    def _start(k, idbuf, xbuf, sem):
        @pl.when(k < _NSUB)
        def _():
            pltpu.async_copy(batch_hbm.at[pl.ds(k * _S, _S)], idbuf, sem)
            pltpu.async_copy(x_hbm.at[pl.ds(k * _S, _S), :], xbuf, sem)

    def _finish(k, idbuf, xbuf, sem):
        @pl.when(k < _NSUB)
        def _():
            pltpu.make_async_copy(batch_hbm.at[pl.ds(k * _S, _S)],
                                  idbuf, sem).wait()
            pltpu.make_async_copy(x_hbm.at[pl.ds(k * _S, _S), :],
                                  xbuf, sem).wait()
            pltpu.sync_copy(xbuf, acc_sh.at[idbuf], add=True)

    _start(wid, ida, xbufa, sema)

    def _round(jj, carry):
        ka = wid + _NW * (2 * jj)
        kb = ka + _NW
        ka2 = ka + 2 * _NW
        _start(kb, idb, xbufb, semb)
        _finish(ka, ida, xbufa, sema)
        _start(ka2, ida, xbufa, sema)
        _finish(kb, idb, xbufb, semb)
        return carry

    lax.fori_loop(0, _HALF, _round, 0)

    plsc.subcore_barrier()
    pltpu.sync_copy(acc_sh.at[pl.ds(sid * 16, 16), :],
                    acc_out.at[cid, pl.ds(sid * 16, 16), :])


_sc_segsum = functools.partial(
    pl.kernel,
    mesh=plsc.VectorSubcoreMesh(core_axis_name="c", subcore_axis_name="s"),
    out_type=jax.ShapeDtypeStruct((2, _B, _D), jnp.float32),
    scratch_types=[
        pltpu.VMEM((_S, _D), jnp.float32),
        pltpu.VMEM((_S, _D), jnp.float32),
        pltpu.VMEM((_S,), jnp.int32),
        pltpu.VMEM((_S,), jnp.int32),
        pltpu.VMEM((16, _D), jnp.float32),
        pltpu.VMEM_SHARED((_B, _D), jnp.float32),
        pltpu.SemaphoreType.DMA,
        pltpu.SemaphoreType.DMA,
    ],
)(_sc_body)


def _onehot_body(batch_ref, x_ref, out_ref, acc):
    i = pl.program_id(0)

    @pl.when(i == 0)
    def _init():
        acc[...] = jnp.zeros_like(acc)

    ids = batch_ref[0, 0, :]
    onehot = (jax.lax.broadcasted_iota(jnp.int32, (_B, _RB), 0)
              == ids[None, :]).astype(jnp.bfloat16)
    xf = x_ref[...]
    x_hi = xf.astype(jnp.bfloat16)
    x_lo = (xf - x_hi.astype(jnp.float32)).astype(jnp.bfloat16)
    acc[...] += (jnp.dot(onehot, x_hi, preferred_element_type=jnp.float32)
                 + jnp.dot(onehot, x_lo, preferred_element_type=jnp.float32))

    @pl.when(i == _NTB - 1)
    def _finish():
        out_ref[...] = acc[...]


def _count_body(batch_ref, cnt_ref):
    ids = batch_ref[0, :]
    hi = ids // 16
    lo = ids - hi * 16
    H = (jax.lax.broadcasted_iota(jnp.int32, (16, _N), 0)
         == hi[None, :]).astype(jnp.float32)
    L = (jax.lax.broadcasted_iota(jnp.int32, (16, _N), 0)
         == lo[None, :]).astype(jnp.float32)
    cnt_ref[...] = jax.lax.dot_general(
        H, L, dimension_numbers=(((1,), (1,)), ((), ())),
        preferred_element_type=jnp.float32)


def _mlp_body(acc_ref, tc_ref, cnt_ref, u_ref, W1_ref, b1_ref, W2_ref,
              b2_ref, out_ref):
    sums = acc_ref[0] + acc_ref[1] + tc_ref[...]
    pooled = sums / jnp.maximum(cnt_ref[...], 1.0)
    h = jnp.maximum(
        jnp.dot(u_ref[...], W1_ref[0:_G, :],
                preferred_element_type=jnp.float32)
        + jnp.dot(pooled, W1_ref[_G:_G + _D, :],
                  preferred_element_type=jnp.float32)
        + b1_ref[...], 0.0)
    out_ref[...] = (jnp.dot(h, W2_ref[...],
                            preferred_element_type=jnp.float32)
                    + b2_ref[...])


def kernel(x, edge_index, edge_attr, u, batch, W1, b1, W2, b2):
    del edge_index, edge_attr
    batch_i32 = batch.astype(jnp.int32)
    acc2 = _sc_segsum(x, batch_i32)

    batch3d_tc = batch_i32[_NSC:].reshape(_NTB, 1, _RB)
    tc_acc = pl.pallas_call(
        _onehot_body,
        grid=(_NTB,),
        in_specs=[
            pl.BlockSpec((1, 1, _RB), lambda i: (i, 0, 0)),
            pl.BlockSpec((_RB, _D), lambda i: (i + _TC0, 0)),
        ],
        out_specs=pl.BlockSpec((_B, _D), lambda i: (0, 0)),
        out_shape=jax.ShapeDtypeStruct((_B, _D), jnp.float32),
        scratch_shapes=[pltpu.VMEM((_B, _D), jnp.float32)],
    )(batch3d_tc, x)

    cnt16 = pl.pallas_call(
        _count_body,
        out_shape=jax.ShapeDtypeStruct((16, 16), jnp.float32),
    )(batch_i32.reshape(1, _N))
    cnt = cnt16.reshape(_B, 1)

    return pl.pallas_call(
        _mlp_body,
        out_shape=jax.ShapeDtypeStruct((_B, _G), jnp.float32),
    )(acc2, tc_acc, cnt, u, W1, b1.reshape(1, _G), W2, b2.reshape(1, _G))


# speedup vs baseline: 1.4020x; 1.4020x over previous
import functools

import jax
import jax.numpy as jnp
from jax import lax
from jax.experimental import pallas as pl
from jax.experimental.pallas import tpu as pltpu
from jax.experimental.pallas import tpu_sc as plsc

_N = 100000
_D = 128
_G = 128
_B = 256

_NSC = 48000
_S = 128
_NSUB = _NSC // _S
_NW = 32
_MAXJ = -(-_NSUB // _NW)
_HALF = _MAXJ // 2

_RB = 4000
_TC0 = _NSC // _RB
_NTB = (_N - _NSC) // _RB


def _sc_body(x_hbm, batch_hbm, acc_out,
             xbufa, xbufb, ida, idb, zrow, acc_sh, sema, semb):
    cid = lax.axis_index("c")
    sid = lax.axis_index("s")
    wid = sid * 2 + cid

    zero16 = jnp.zeros((16,), jnp.float32)

    def _init_z(i, carry):
        for g in range(_D // 16):
            zrow[i, pl.ds(g * 16, 16)] = zero16
        return carry

    lax.fori_loop(0, 16, _init_z, 0)

    pltpu.sync_copy(zrow, acc_sh.at[pl.ds(sid * 16, 16), :])
    plsc.subcore_barrier()

    def _start(k, idbuf, xbuf, sem):
        @pl.when(k < _NSUB)
        def _():
            pltpu.async_copy(batch_hbm.at[pl.ds(k * _S, _S)], idbuf, sem)
            pltpu.async_copy(x_hbm.at[pl.ds(k * _S, _S), :], xbuf, sem)

    def _finish(k, idbuf, xbuf, sem):
        @pl.when(k < _NSUB)
        def _():
            pltpu.make_async_copy(batch_hbm.at[pl.ds(k * _S, _S)],
                                  idbuf, sem).wait()
            pltpu.make_async_copy(x_hbm.at[pl.ds(k * _S, _S), :],
                                  xbuf, sem).wait()
            pltpu.sync_copy(xbuf, acc_sh.at[idbuf], add=True)

    _start(wid, ida, xbufa, sema)

    def _round(jj, carry):
        ka = wid + _NW * (2 * jj)
        kb = ka + _NW
        ka2 = ka + 2 * _NW
        _start(kb, idb, xbufb, semb)
        _finish(ka, ida, xbufa, sema)
        _start(ka2, ida, xbufa, sema)
        _finish(kb, idb, xbufb, semb)
        return carry

    lax.fori_loop(0, _HALF, _round, 0)

    plsc.subcore_barrier()
    pltpu.sync_copy(acc_sh.at[pl.ds(sid * 16, 16), :],
                    acc_out.at[cid, pl.ds(sid * 16, 16), :])


_sc_segsum = functools.partial(
    pl.kernel,
    mesh=plsc.VectorSubcoreMesh(core_axis_name="c", subcore_axis_name="s"),
    out_type=jax.ShapeDtypeStruct((2, _B, _D), jnp.float32),
    scratch_types=[
        pltpu.VMEM((_S, _D), jnp.float32),
        pltpu.VMEM((_S, _D), jnp.float32),
        pltpu.VMEM((_S,), jnp.int32),
        pltpu.VMEM((_S,), jnp.int32),
        pltpu.VMEM((16, _D), jnp.float32),
        pltpu.VMEM_SHARED((_B, _D), jnp.float32),
        pltpu.SemaphoreType.DMA,
        pltpu.SemaphoreType.DMA,
    ],
)(_sc_body)


def _onehot_body(batch_ref, x_ref, out_ref, acc):
    i = pl.program_id(0)

    @pl.when(i == 0)
    def _init():
        acc[...] = jnp.zeros_like(acc)

    ids = batch_ref[0, 0, :]
    onehot = (jax.lax.broadcasted_iota(jnp.int32, (_B, _RB), 0)
              == ids[None, :]).astype(jnp.bfloat16)
    xf = x_ref[...]
    x_hi = xf.astype(jnp.bfloat16)
    x_lo = (xf - x_hi.astype(jnp.float32)).astype(jnp.bfloat16)
    acc[...] += (jnp.dot(onehot, x_hi, preferred_element_type=jnp.float32)
                 + jnp.dot(onehot, x_lo, preferred_element_type=jnp.float32))

    @pl.when(i == _NTB - 1)
    def _finish():
        out_ref[...] = acc[...]


def _count_body(batch_ref, cnt_ref):
    ids = batch_ref[0, :]
    hi = ids // 16
    lo = ids - hi * 16
    H = (jax.lax.broadcasted_iota(jnp.int32, (16, _N), 0)
         == hi[None, :]).astype(jnp.float32)
    L = (jax.lax.broadcasted_iota(jnp.int32, (16, _N), 0)
         == lo[None, :]).astype(jnp.float32)
    cnt_ref[...] = jax.lax.dot_general(
        H, L, dimension_numbers=(((1,), (1,)), ((), ())),
        preferred_element_type=jnp.float32)


def _mlp_body(acc_ref, tc_ref, cnt_ref, u_ref, W1_ref, b1_ref, W2_ref,
              b2_ref, out_ref):
    sums = acc_ref[0] + acc_ref[1] + tc_ref[...]
    pooled = sums / jnp.maximum(cnt_ref[...], 1.0)
    h = jnp.maximum(
        jnp.dot(u_ref[...], W1_ref[0:_G, :],
                preferred_element_type=jnp.float32)
        + jnp.dot(pooled, W1_ref[_G:_G + _D, :],
                  preferred_element_type=jnp.float32)
        + b1_ref[...], 0.0)
    out_ref[...] = (jnp.dot(h, W2_ref[...],
                            preferred_element_type=jnp.float32)
                    + b2_ref[...])


def kernel(x, edge_index, edge_attr, u, batch, W1, b1, W2, b2):
    del edge_index, edge_attr
    batch_i32 = batch.astype(jnp.int32)
    acc2 = _sc_segsum(x, batch_i32)

    batch3d_tc = batch_i32[_NSC:].reshape(_NTB, 1, _RB)
    tc_acc = pl.pallas_call(
        _onehot_body,
        grid=(_NTB,),
        in_specs=[
            pl.BlockSpec((1, 1, _RB), lambda i: (i, 0, 0)),
            pl.BlockSpec((_RB, _D), lambda i: (i + _TC0, 0)),
        ],
        out_specs=pl.BlockSpec((_B, _D), lambda i: (0, 0)),
        out_shape=jax.ShapeDtypeStruct((_B, _D), jnp.float32),
        scratch_shapes=[pltpu.VMEM((_B, _D), jnp.float32)],
    )(batch3d_tc, x)

    cnt16 = pl.pallas_call(
        _count_body,
        out_shape=jax.ShapeDtypeStruct((16, 16), jnp.float32),
    )(batch_i32.reshape(1, _N))
    cnt = cnt16.reshape(_B, 1)

    return pl.pallas_call(
        _mlp_body,
        out_shape=jax.ShapeDtypeStruct((_B, _G), jnp.float32),
    )(acc2, tc_acc, cnt, u, W1, b1.reshape(1, _G), W2, b2.reshape(1, _G))


# speedup vs baseline: 1.5312x; 1.0922x over previous
import functools

import jax
import jax.numpy as jnp
from jax import lax
from jax.experimental import pallas as pl
from jax.experimental.pallas import tpu as pltpu
from jax.experimental.pallas import tpu_sc as plsc

_N = 100000
_D = 128
_G = 128
_B = 256

_NSC = 64000
_S = 128
_NSUB = _NSC // _S
_NW = 32
_MAXJ = -(-_NSUB // _NW)
_HALF = _MAXJ // 2

_RB = 4000
_TC0 = _NSC // _RB
_NTB = (_N - _NSC) // _RB


def _sc_body(x_hbm, batch_hbm, acc_out,
             xbufa, xbufb, ida, idb, zrow, acc_sh, sema, semb):
    cid = lax.axis_index("c")
    sid = lax.axis_index("s")
    wid = sid * 2 + cid

    zero16 = jnp.zeros((16,), jnp.float32)

    def _init_z(i, carry):
        for g in range(_D // 16):
            zrow[i, pl.ds(g * 16, 16)] = zero16
        return carry

    lax.fori_loop(0, 16, _init_z, 0)

    pltpu.sync_copy(zrow, acc_sh.at[pl.ds(sid * 16, 16), :])
    plsc.subcore_barrier()

    def _start(k, idbuf, xbuf, sem):
        @pl.when(k < _NSUB)
        def _():
            pltpu.async_copy(batch_hbm.at[pl.ds(k * _S, _S)], idbuf, sem)
            pltpu.async_copy(x_hbm.at[pl.ds(k * _S, _S), :], xbuf, sem)

    def _finish(k, idbuf, xbuf, sem):
        @pl.when(k < _NSUB)
        def _():
            pltpu.make_async_copy(batch_hbm.at[pl.ds(k * _S, _S)],
                                  idbuf, sem).wait()
            pltpu.make_async_copy(x_hbm.at[pl.ds(k * _S, _S), :],
                                  xbuf, sem).wait()
            pltpu.sync_copy(xbuf, acc_sh.at[idbuf], add=True)

    _start(wid, ida, xbufa, sema)

    def _round(jj, carry):
        ka = wid + _NW * (2 * jj)
        kb = ka + _NW
        ka2 = ka + 2 * _NW
        _start(kb, idb, xbufb, semb)
        _finish(ka, ida, xbufa, sema)
        _start(ka2, ida, xbufa, sema)
        _finish(kb, idb, xbufb, semb)
        return carry

    lax.fori_loop(0, _HALF, _round, 0)

    plsc.subcore_barrier()
    pltpu.sync_copy(acc_sh.at[pl.ds(sid * 16, 16), :],
                    acc_out.at[cid, pl.ds(sid * 16, 16), :])


_sc_segsum = functools.partial(
    pl.kernel,
    mesh=plsc.VectorSubcoreMesh(core_axis_name="c", subcore_axis_name="s"),
    out_type=jax.ShapeDtypeStruct((2, _B, _D), jnp.float32),
    scratch_types=[
        pltpu.VMEM((_S, _D), jnp.float32),
        pltpu.VMEM((_S, _D), jnp.float32),
        pltpu.VMEM((_S,), jnp.int32),
        pltpu.VMEM((_S,), jnp.int32),
        pltpu.VMEM((16, _D), jnp.float32),
        pltpu.VMEM_SHARED((_B, _D), jnp.float32),
        pltpu.SemaphoreType.DMA,
        pltpu.SemaphoreType.DMA,
    ],
)(_sc_body)


def _onehot_body(batch_ref, x_ref, out_ref, acc):
    i = pl.program_id(0)

    @pl.when(i == 0)
    def _init():
        acc[...] = jnp.zeros_like(acc)

    ids = batch_ref[0, 0, :]
    onehot = (jax.lax.broadcasted_iota(jnp.int32, (_B, _RB), 0)
              == ids[None, :]).astype(jnp.bfloat16)
    xf = x_ref[...]
    x_hi = xf.astype(jnp.bfloat16)
    x_lo = (xf - x_hi.astype(jnp.float32)).astype(jnp.bfloat16)
    acc[...] += (jnp.dot(onehot, x_hi, preferred_element_type=jnp.float32)
                 + jnp.dot(onehot, x_lo, preferred_element_type=jnp.float32))

    @pl.when(i == _NTB - 1)
    def _finish():
        out_ref[...] = acc[...]


def _count_body(batch_ref, cnt_ref):
    ids = batch_ref[0, :]
    hi = ids // 16
    lo = ids - hi * 16
    H = (jax.lax.broadcasted_iota(jnp.int32, (16, _N), 0)
         == hi[None, :]).astype(jnp.float32)
    L = (jax.lax.broadcasted_iota(jnp.int32, (16, _N), 0)
         == lo[None, :]).astype(jnp.float32)
    cnt_ref[...] = jax.lax.dot_general(
        H, L, dimension_numbers=(((1,), (1,)), ((), ())),
        preferred_element_type=jnp.float32)


def _mlp_body(acc_ref, tc_ref, cnt_ref, u_ref, W1_ref, b1_ref, W2_ref,
              b2_ref, out_ref):
    sums = acc_ref[0] + acc_ref[1] + tc_ref[...]
    pooled = sums / jnp.maximum(cnt_ref[...], 1.0)
    h = jnp.maximum(
        jnp.dot(u_ref[...], W1_ref[0:_G, :],
                preferred_element_type=jnp.float32)
        + jnp.dot(pooled, W1_ref[_G:_G + _D, :],
                  preferred_element_type=jnp.float32)
        + b1_ref[...], 0.0)
    out_ref[...] = (jnp.dot(h, W2_ref[...],
                            preferred_element_type=jnp.float32)
                    + b2_ref[...])


def kernel(x, edge_index, edge_attr, u, batch, W1, b1, W2, b2):
    del edge_index, edge_attr
    batch_i32 = batch.astype(jnp.int32)
    acc2 = _sc_segsum(x, batch_i32)

    batch3d_tc = batch_i32[_NSC:].reshape(_NTB, 1, _RB)
    tc_acc = pl.pallas_call(
        _onehot_body,
        grid=(_NTB,),
        in_specs=[
            pl.BlockSpec((1, 1, _RB), lambda i: (i, 0, 0)),
            pl.BlockSpec((_RB, _D), lambda i: (i + _TC0, 0)),
        ],
        out_specs=pl.BlockSpec((_B, _D), lambda i: (0, 0)),
        out_shape=jax.ShapeDtypeStruct((_B, _D), jnp.float32),
        scratch_shapes=[pltpu.VMEM((_B, _D), jnp.float32)],
    )(batch3d_tc, x)

    cnt16 = pl.pallas_call(
        _count_body,
        out_shape=jax.ShapeDtypeStruct((16, 16), jnp.float32),
    )(batch_i32.reshape(1, _N))
    cnt = cnt16.reshape(_B, 1)

    return pl.pallas_call(
        _mlp_body,
        out_shape=jax.ShapeDtypeStruct((_B, _G), jnp.float32),
    )(acc2, tc_acc, cnt, u, W1, b1.reshape(1, _G), W2, b2.reshape(1, _G))
